# Initial kernel scaffold; baseline (speedup 1.0000x reference)
import jax
import jax.numpy as jnp
from jax.experimental import pallas as pl


def kernel(x, edge_index, x_batch, Wl1, bl1, Wr1, Wl2, bl2, Wr2):
    raise NotImplementedError("write your pallas kernel here")


# speedup vs baseline: 5.5277x; 5.5277x over previous
import dataclasses
import functools

import jax
import jax.numpy as jnp
from jax import lax
from jax.experimental import pallas as pl
from jax.experimental.pallas import tpu as pltpu
from jax.experimental.pallas import tpu_sc as plsc

_N = 10000
_E = 320000
_D = 128
_NC = 2
_NS = 16
_NW = _NC * _NS
_EW = _E // _NW
_K = 80
_NCHUNK = _EW // _K
_ZR = 40
_NB = _N // _ZR


def _build_sc_agg(with_counts: bool):
    mesh = plsc.VectorSubcoreMesh(core_axis_name="c", subcore_axis_name="s")
    out_type = [jax.ShapeDtypeStruct((_NC, _N, _D), jnp.float32)]
    scratch = [
        pltpu.VMEM((_K,), jnp.int32),
        pltpu.VMEM((_K,), jnp.int32),
        pltpu.VMEM((_K, _D), jnp.float32),
        pltpu.VMEM((_ZR, _D), jnp.float32),
        pltpu.VMEM_SHARED((_N, _D), jnp.float32),
        pltpu.SemaphoreType.DMA,
    ]
    if with_counts:
        out_type.append(jax.ShapeDtypeStruct((_NW * _N,), jnp.float32))
        scratch.append(pltpu.VMEM((_N,), jnp.float32))

    def body(x_hbm, src_hbm, dst_hbm, acc_out, *rest):
        if with_counts:
            cnt_out, src_v, dst_v, rows_v, zero_v, acc_sh, sem, cnt_v = rest
        else:
            src_v, dst_v, rows_v, zero_v, acc_sh, sem = rest
        cid = lax.axis_index("c")
        sid = lax.axis_index("s")
        wid = cid * _NS + sid

        zeros16 = jnp.zeros((16,), jnp.float32)
        ones16 = jnp.ones((16,), jnp.float32)

        @pl.loop(0, _ZR)
        def _(r):
            @pl.loop(0, _D, step=16)
            def _(cc):
                zero_v[r, pl.ds(cc, 16)] = zeros16

        @pl.loop(sid, _NB, step=_NS)
        def _(b):
            off = pl.multiple_of(b * _ZR, 8)
            pltpu.sync_copy(zero_v, acc_sh.at[pl.ds(off, _ZR)])

        if with_counts:
            @pl.loop(0, _N, step=16)
            def _(i):
                cnt_v[pl.ds(i, 16)] = zeros16

        plsc.subcore_barrier()

        ebase = wid * _EW

        @pl.loop(0, _NCHUNK)
        def _(c):
            base = ebase + c * _K
            pltpu.sync_copy(src_hbm.at[pl.ds(base, _K)], src_v)
            pltpu.sync_copy(dst_hbm.at[pl.ds(base, _K)], dst_v)
            pltpu.async_copy(x_hbm.at[src_v], rows_v, sem).wait()
            pltpu.sync_copy(rows_v, acc_sh.at[dst_v], add=True)
            if with_counts:
                @pl.loop(0, _K, step=16)
                def _(j):
                    idx16 = dst_v[pl.ds(j, 16)]
                    plsc.addupdate_scatter(cnt_v, [idx16], ones16)

        plsc.subcore_barrier()

        @pl.loop(sid, _NB, step=_NS)
        def _(b):
            off = pl.multiple_of(b * _ZR, 8)
            pltpu.sync_copy(acc_sh.at[pl.ds(off, _ZR)],
                            acc_out.at[cid, pl.ds(off, _ZR)])
        if with_counts:
            pltpu.sync_copy(cnt_v, cnt_out.at[pl.ds(wid * _N, _N)])

    cp = pltpu.CompilerParams()
    if "needs_layout_passes" in pltpu.CompilerParams.__dataclass_fields__:
        cp = dataclasses.replace(cp, needs_layout_passes=False)
    return pl.kernel(
        body,
        out_type=tuple(out_type) if with_counts else out_type[0],
        mesh=mesh,
        scratch_types=scratch,
        compiler_params=cp,
    )


_sc_agg_counts = _build_sc_agg(with_counts=True)
_sc_agg = _build_sc_agg(with_counts=False)


def _tc_dense(acc, cnt, x, Wl, bl, Wr):
    def body(acc_ref, cnt_ref, x_ref, wl_ref, bl_ref, wr_ref, h_ref):
        total = jnp.sum(cnt_ref[...], axis=0)
        inv = 1.0 / jnp.maximum(total, 1.0)
        mean = (acc_ref[0] + acc_ref[1]) * inv[:, None]
        dn = (((1,), (1,)), ((), ()))
        h = lax.dot_general(mean, wl_ref[...], dn,
                            precision=lax.Precision.HIGHEST,
                            preferred_element_type=jnp.float32)
        h = h + bl_ref[...][None, :]
        h = h + lax.dot_general(x_ref[...], wr_ref[...], dn,
                                precision=lax.Precision.HIGHEST,
                                preferred_element_type=jnp.float32)
        h_ref[...] = h

    return pl.pallas_call(
        body,
        out_shape=jax.ShapeDtypeStruct((_N, _D), jnp.float32),
    )(acc, cnt, x, Wl, bl, Wr)


@jax.jit
def _impl(x, edge_index, x_batch, Wl1, bl1, Wr1, Wl2, bl2, Wr2):
    src = edge_index[0]
    dst = edge_index[1]
    acc1, cnt_flat = _sc_agg_counts(x, src, dst)
    cnt = cnt_flat.reshape(_NW, _N)
    h = _tc_dense(acc1, cnt, x, Wl1, bl1, Wr1)
    acc2 = _sc_agg(h, src, dst)
    return _tc_dense(acc2, cnt, h, Wl2, bl2, Wr2)


def kernel(x, edge_index, x_batch, Wl1, bl1, Wr1, Wl2, bl2, Wr2):
    return _impl(x, edge_index, x_batch, Wl1, bl1, Wr1, Wl2, bl2, Wr2)


# speedup vs baseline: 12.1617x; 2.2001x over previous
import dataclasses
import functools

import jax
import jax.numpy as jnp
from jax import lax
from jax.experimental import pallas as pl
from jax.experimental.pallas import tpu as pltpu
from jax.experimental.pallas import tpu_sc as plsc

_N = 10000
_E = 320000
_D = 128
_NC = 2
_NS = 16
_NW = _NC * _NS
_EW = _E // _NW
_ZR = 8
_ZB = _N // _ZR
_WR = 40
_WB = _N // _WR


def _build_sc_agg(with_counts: bool, k: int):
    nchunk = _EW // k
    assert nchunk % 4 == 1 and nchunk >= 9
    mesh = plsc.VectorSubcoreMesh(core_axis_name="c", subcore_axis_name="s")
    out_type = [jax.ShapeDtypeStruct((_NC, _N, _D), jnp.float32)]
    scratch = (
        [pltpu.VMEM((k,), jnp.int32) for _ in range(4)]
        + [pltpu.VMEM((k,), jnp.int32) for _ in range(4)]
        + [pltpu.VMEM((k, _D), jnp.float32) for _ in range(2)]
        + [
            pltpu.VMEM((_ZR, _D), jnp.float32),
            pltpu.VMEM_SHARED((_N, _D), jnp.float32),
        ]
        + [pltpu.SemaphoreType.DMA for _ in range(8)]
    )
    if with_counts:
        out_type.append(jax.ShapeDtypeStruct((_NW * _N,), jnp.float32))
        scratch.append(pltpu.VMEM((_N,), jnp.float32))

    def body(x_hbm, src_hbm, dst_hbm, acc_out, *rest):
        if with_counts:
            cnt_out = rest[0]
            rest = rest[1:]
            cnt_v = rest[-1]
            rest = rest[:-1]
        srcv = rest[0:4]
        dstv = rest[4:8]
        rows = rest[8:10]
        zero_v = rest[10]
        acc_sh = rest[11]
        isem = rest[12:16]
        gsem = rest[16:18]
        ssem = rest[18:20]
        cid = lax.axis_index("c")
        sid = lax.axis_index("s")
        wid = cid * _NS + sid
        ebase = wid * _EW

        zeros16 = jnp.zeros((16,), jnp.float32)
        ones16 = jnp.ones((16,), jnp.float32)

        def issue_idx(c, j):
            base = ebase + c * k
            pltpu.async_copy(src_hbm.at[pl.ds(base, k)], srcv[j], isem[j])
            pltpu.async_copy(dst_hbm.at[pl.ds(base, k)], dstv[j], isem[j])

        def wait_idx(c, j):
            base = ebase + c * k
            pltpu.make_async_copy(
                src_hbm.at[pl.ds(base, k)], srcv[j], isem[j]).wait()
            pltpu.make_async_copy(
                dst_hbm.at[pl.ds(base, k)], dstv[j], isem[j]).wait()

        for j in range(4):
            issue_idx(j, j)

        @pl.loop(0, _ZR)
        def _(r):
            @pl.loop(0, _D, step=16)
            def _(cc):
                zero_v[r, pl.ds(cc, 16)] = zeros16

        @pl.loop(sid, _ZB, step=_NS)
        def _(b):
            off = pl.multiple_of(b * _ZR, 8)
            pltpu.sync_copy(zero_v, acc_sh.at[pl.ds(off, _ZR)])

        if with_counts:
            @pl.loop(0, _N, step=16)
            def _(i):
                cnt_v[pl.ds(i, 16)] = zeros16

        plsc.subcore_barrier()

        def issue_gather(c, b2, b4):
            pltpu.async_copy(x_hbm.at[srcv[b4]], rows[b2], gsem[b2])

        def do_counts(b4):
            @pl.loop(0, k, step=16)
            def _(j):
                idx16 = dstv[b4][pl.ds(j, 16)]
                plsc.addupdate_scatter(cnt_v, [idx16], ones16)

        def step(c, b2, b4, more_g, more_i):
            pltpu.make_async_copy(
                x_hbm.at[srcv[b4]], rows[b2], gsem[b2]).wait()
            pltpu.async_copy(rows[b2], acc_sh.at[dstv[b4]], ssem[b2],
                             add=True)
            if with_counts:
                do_counts(b4)
            pltpu.make_async_copy(
                rows[b2], acc_sh.at[dstv[b4]], ssem[b2]).wait()
            if more_i:
                issue_idx(c + 4, b4)
            if more_g:
                wait_idx(c + 2, (b4 + 2) % 4)
                issue_gather(c + 2, b2, (b4 + 2) % 4)

        wait_idx(0, 0)
        issue_gather(0, 0, 0)
        wait_idx(1, 1)
        issue_gather(1, 1, 1)

        main_end = ((nchunk // 4) - 1) * 4

        @pl.loop(0, main_end, step=4)
        def _(c):
            for u in range(4):
                step(c + u, u % 2, u, True, True)

        for c in range(main_end, nchunk):
            step(c, c % 2, c % 4, c + 2 < nchunk, c + 4 < nchunk)

        plsc.subcore_barrier()

        @pl.loop(sid, _WB, step=_NS)
        def _(b):
            off = pl.multiple_of(b * _WR, 8)
            pltpu.sync_copy(acc_sh.at[pl.ds(off, _WR)],
                            acc_out.at[cid, pl.ds(off, _WR)])
        if with_counts:
            pltpu.sync_copy(cnt_v, cnt_out.at[pl.ds(wid * _N, _N)])

    cp = pltpu.CompilerParams()
    if "needs_layout_passes" in pltpu.CompilerParams.__dataclass_fields__:
        cp = dataclasses.replace(cp, needs_layout_passes=False)
    return pl.kernel(
        body,
        out_type=tuple(out_type) if with_counts else out_type[0],
        mesh=mesh,
        scratch_types=scratch,
        compiler_params=cp,
    )


_K1 = 80
_K2 = 80
_sc_agg_counts = _build_sc_agg(with_counts=True, k=_K1)
_sc_agg = _build_sc_agg(with_counts=False, k=_K2)


def _tc_dense(acc, cnt, x, Wl, bl, Wr):
    def body(acc_ref, cnt_ref, x_ref, wl_ref, bl_ref, wr_ref, h_ref):
        total = jnp.sum(cnt_ref[...], axis=0)
        inv = 1.0 / jnp.maximum(total, 1.0)
        mean = (acc_ref[0] + acc_ref[1]) * inv[:, None]
        dn = (((1,), (1,)), ((), ()))
        h = lax.dot_general(mean, wl_ref[...], dn,
                            precision=lax.Precision.HIGHEST,
                            preferred_element_type=jnp.float32)
        h = h + bl_ref[...][None, :]
        h = h + lax.dot_general(x_ref[...], wr_ref[...], dn,
                                precision=lax.Precision.HIGHEST,
                                preferred_element_type=jnp.float32)
        h_ref[...] = h

    return pl.pallas_call(
        body,
        out_shape=jax.ShapeDtypeStruct((_N, _D), jnp.float32),
    )(acc, cnt, x, Wl, bl, Wr)


@jax.jit
def _impl(x, edge_index, x_batch, Wl1, bl1, Wr1, Wl2, bl2, Wr2):
    src = edge_index[0]
    dst = edge_index[1]
    acc1, cnt_flat = _sc_agg_counts(x, src, dst)
    cnt = cnt_flat.reshape(_NW, _N)
    h = _tc_dense(acc1, cnt, x, Wl1, bl1, Wr1)
    acc2 = _sc_agg(h, src, dst)
    return _tc_dense(acc2, cnt, h, Wl2, bl2, Wr2)


def kernel(x, edge_index, x_batch, Wl1, bl1, Wr1, Wl2, bl2, Wr2):
    return _impl(x, edge_index, x_batch, Wl1, bl1, Wr1, Wl2, bl2, Wr2)
